# Initial kernel scaffold; baseline (speedup 1.0000x reference)
import jax
import jax.numpy as jnp
from jax.experimental import pallas as pl


def kernel(feat, edge_index, W_neigh, b_neigh):
    raise NotImplementedError("write your pallas kernel here")


# speedup vs baseline: 16.4258x; 16.4258x over previous
import jax
import jax.numpy as jnp
from jax import lax
from jax.experimental import pallas as pl
from jax.experimental.pallas import tpu as pltpu
from jax.experimental.pallas import tpu_sc as plsc

N = 10000
E = 320000
D = 128
NC = 2
NS = 16
NW = NC * NS
EPT = E // NW
K = 80
NCH = EPT // K
N_PAD = 10240
ROWS_PT = N_PAD // NS
DEG_PT = ROWS_PT


NI = 6
NB = 4


def _sc_body(feat_h, edge_h, psum_h, deg_h,
             acc, deg,
             src0, src1, src2, src3, src4, src5,
             dst0, dst1, dst2, dst3, dst4, dst5,
             rows0, rows1, rows2, rows3, ones_v, zdeg_v,
             gsem0, gsem1, gsem2, gsem3, ssem0, ssem1, ssem2, ssem3,
             isem0, isem1, isem2, isem3, isem4, isem5):
  c = lax.axis_index("c")
  s = lax.axis_index("s")
  w = c * NS + s
  src_h = edge_h.at[0, 0]
  dst_h = edge_h.at[1, 0]

  def zero_rows(i, carry):
    for j in range(D // 16):
      rows0[i, pl.ds(j * 16, 16)] = jnp.zeros((16,), jnp.float32)
    return carry

  lax.fori_loop(0, K, zero_rows, 0)
  for i in range(DEG_PT // 16):
    zdeg_v[pl.ds(i * 16, 16)] = jnp.zeros((16,), jnp.float32)
  for i in range(K // 16):
    ones_v[pl.ds(i * 16, 16)] = jnp.ones((16,), jnp.float32)
  for i in range(ROWS_PT // K):
    pltpu.async_copy(rows0, acc.at[pl.ds(s * ROWS_PT + i * K, K)], isem0)
  pltpu.sync_copy(zdeg_v, deg.at[pl.ds(s * DEG_PT, DEG_PT)])
  for i in range(ROWS_PT // K):
    pltpu.make_async_copy(rows0, acc.at[pl.ds(s * ROWS_PT + i * K, K)],
                          isem0).wait()
  plsc.subcore_barrier()

  base = w * EPT
  srcs = (src0, src1, src2, src3, src4, src5)
  dsts = (dst0, dst1, dst2, dst3, dst4, dst5)
  rowss = (rows0, rows1, rows2, rows3)
  gsems = (gsem0, gsem1, gsem2, gsem3)
  ssems = (ssem0, ssem1, ssem2, ssem3)
  isems = (isem0, isem1, isem2, isem3, isem4, isem5)

  def fire_idx(g, j):
    off = base + g * K
    pltpu.async_copy(src_h.at[pl.ds(off, K)], srcs[j], isems[j])
    pltpu.async_copy(dst_h.at[pl.ds(off, K)], dsts[j], isems[j])

  def wait_idx_fire_gather(g, j, b):
    off = base + g * K
    pltpu.make_async_copy(src_h.at[pl.ds(off, K)], srcs[j], isems[j]).wait()
    pltpu.make_async_copy(dst_h.at[pl.ds(off, K)], dsts[j], isems[j]).wait()
    pltpu.async_copy(feat_h.at[srcs[j]], rowss[b], gsems[b])

  def wait_scatter(b, j):
    pltpu.make_async_copy(rowss[b], acc.at[dsts[j]], ssems[b]).wait()
    pltpu.make_async_copy(ones_v, deg.at[dsts[j]], ssems[b]).wait()

  def step(g, b, j):
    pltpu.make_async_copy(feat_h.at[srcs[j]], rowss[b], gsems[b]).wait()

    @pl.when(g >= 2)
    def _():
      wait_scatter((b + 2) % NB, (j + 4) % NI)

    pltpu.async_copy(rowss[b], acc.at[dsts[j]], ssems[b], add=True)
    pltpu.async_copy(ones_v, deg.at[dsts[j]], ssems[b], add=True)

    @pl.when(g + 2 < NCH)
    def _():
      wait_idx_fire_gather(g + 2, (j + 2) % NI, (b + 2) % NB)

    @pl.when(g + 3 < NCH)
    def _():
      fire_idx(g + 3, (j + 3) % NI)

  fire_idx(0, 0)
  fire_idx(1, 1)
  fire_idx(2, 2)
  wait_idx_fire_gather(0, 0, 0)
  wait_idx_fire_gather(1, 1, 1)

  NU = 12

  def twelve_steps(i, carry):
    for u in range(NU):
      step(NU * i + u, u % NB, u % NI)
    return carry

  lax.fori_loop(0, NCH // NU, twelve_steps, 0)
  for g in range(NCH - NCH % NU, NCH):
    step(g, g % NB, g % NI)
  wait_scatter((NCH - 2) % NB, (NCH - 2) % NI)
  wait_scatter((NCH - 1) % NB, (NCH - 1) % NI)
  plsc.subcore_barrier()

  pltpu.sync_copy(acc.at[pl.ds(s * ROWS_PT, ROWS_PT)],
                  psum_h.at[c, pl.ds(s * ROWS_PT, ROWS_PT)])
  pltpu.sync_copy(deg.at[pl.ds(s * DEG_PT, DEG_PT)],
                  deg_h.at[c, pl.ds(s * DEG_PT, DEG_PT)])


@jax.jit
def _sc_aggregate(feat, edge3):
  return pl.kernel(
      _sc_body,
      out_type=(
          jax.ShapeDtypeStruct((NC, N_PAD, D), jnp.float32),
          jax.ShapeDtypeStruct((NC, N_PAD), jnp.float32),
      ),
      mesh=plsc.VectorSubcoreMesh(core_axis_name="c", subcore_axis_name="s"),
      scratch_types=(
          [pltpu.VMEM_SHARED((N_PAD, D), jnp.float32),
           pltpu.VMEM_SHARED((N_PAD,), jnp.float32)]
          + [pltpu.VMEM((K,), jnp.int32)] * 12
          + [pltpu.VMEM((K, D), jnp.float32)] * 4
          + [pltpu.VMEM((K,), jnp.float32),
             pltpu.VMEM((DEG_PT,), jnp.float32)]
          + [pltpu.SemaphoreType.DMA] * 14
      ),
  )(feat, edge3)


BLK = 1000


def _tc_body(p_ref, d_ref, feat_ref, w_ref, b_ref, o_ref):
  nsum = p_ref[0] + p_ref[1]
  degs = d_ref[0] + d_ref[1]
  h_neigh = nsum / jnp.maximum(degs, 1.0)
  dims = (((1,), (1,)), ((), ()))
  o_ref[...] = (
      lax.dot_general(feat_ref[...], w_ref[:, :D], dims,
                      preferred_element_type=jnp.float32)
      + lax.dot_general(h_neigh, w_ref[:, D:], dims,
                        preferred_element_type=jnp.float32)
      + b_ref[...]
  )


@jax.jit
def _tc_combine(psum, deg3, feat, w, b2):
  return pl.pallas_call(
      _tc_body,
      grid=(N // BLK,),
      in_specs=[
          pl.BlockSpec((NC, BLK, D), lambda i: (0, i, 0)),
          pl.BlockSpec((NC, BLK, 1), lambda i: (0, i, 0)),
          pl.BlockSpec((BLK, D), lambda i: (i, 0)),
          pl.BlockSpec((D, 2 * D), lambda i: (0, 0)),
          pl.BlockSpec((1, D), lambda i: (0, 0)),
      ],
      out_specs=pl.BlockSpec((BLK, D), lambda i: (i, 0)),
      out_shape=jax.ShapeDtypeStruct((N, D), jnp.float32),
  )(psum, deg3, feat, w, b2)


def kernel(feat, edge_index, W_neigh, b_neigh):
  edge3 = edge_index.astype(jnp.int32).reshape(2, 1, E)
  psum, degp = _sc_aggregate(feat, edge3)
  return _tc_combine(psum, degp.reshape(NC, N_PAD, 1), feat, W_neigh,
                     b_neigh.reshape(1, D))
